# Initial kernel scaffold; baseline (speedup 1.0000x reference)
import jax
import jax.numpy as jnp
from jax.experimental import pallas as pl


def kernel(image, boxes, box_ind):
    raise NotImplementedError("write your pallas kernel here")


# speedup vs baseline: 2.3471x; 2.3471x over previous
import jax
import jax.numpy as jnp
from jax.experimental import pallas as pl
from jax.experimental.pallas import tpu as pltpu

_CROP = 24


def _axis(lo, hi, size, crop):
    scale = (hi - lo) * (size - 1.0) / (crop - 1.0)
    c = lo[:, None] * (size - 1.0) + (
        jnp.arange(crop, dtype=jnp.float32)[None, :] * scale[:, None]
    )
    valid = (c >= 0.0) & (c <= size - 1.0)
    c0 = jnp.floor(c)
    frac = c - c0
    i0 = jnp.clip(c0.astype(jnp.int32), 0, size - 1)
    i1 = jnp.clip(i0 + 1, 0, size - 1)
    return i0, i1, frac, valid


def _weight_matrix(i0, i1, frac, valid, size):
    oh0 = jax.nn.one_hot(i0, size, dtype=jnp.float32)
    oh1 = jax.nn.one_hot(i1, size, dtype=jnp.float32)
    w = oh0 * (1.0 - frac)[..., None] + oh1 * frac[..., None]
    return w * valid.astype(jnp.float32)[..., None]


def _interp_kernel(bi_ref, z0_ref, z1_ref, wz_ref, wy_ref, wxt_ref,
                   p0_ref, p1_ref, out_ref):
    b = pl.program_id(0)
    z = pl.program_id(1)
    p0 = p0_ref[0, :, 0, :, :]
    p1 = p1_ref[0, :, 0, :, :]
    p = p0 * wz_ref[b, z, 0] + p1 * wz_ref[b, z, 1]
    s = jnp.dot(p.reshape(32 * 64, 64), wxt_ref[0],
                preferred_element_type=jnp.float32)
    s3 = s.reshape(32, 64, 24)
    q = jax.lax.dot_general(wy_ref[0], s3, (((1,), (1,)), ((), ())),
                            preferred_element_type=jnp.float32)
    out_ref[0, 0] = q


def kernel(image, boxes, box_ind):
    n, c, d, h, w = image.shape
    bz1, by1, bx1, bz2, by2, bx2 = (boxes[:, i] for i in range(6))
    z0, z1, fz, vz = _axis(bz1, bz2, d, _CROP)
    y0, y1, fy, vy = _axis(by1, by2, h, _CROP)
    x0, x1, fx, vx = _axis(bx1, bx2, w, _CROP)

    vzf = vz.astype(jnp.float32)
    wz = jnp.stack([(1.0 - fz) * vzf, fz * vzf], axis=-1)
    wy = _weight_matrix(y0, y1, fy, vy, h)
    wxt = jnp.transpose(_weight_matrix(x0, x1, fx, vx, w), (0, 2, 1))

    bi = box_ind.astype(jnp.int32)
    nb = boxes.shape[0]

    grid_spec = pltpu.PrefetchScalarGridSpec(
        num_scalar_prefetch=4,
        grid=(nb, _CROP),
        in_specs=[
            pl.BlockSpec((1, _CROP, h), lambda b, z, *sp: (b, 0, 0)),
            pl.BlockSpec((1, w, _CROP), lambda b, z, *sp: (b, 0, 0)),
            pl.BlockSpec((1, c, 1, h, w),
                         lambda b, z, bi_, z0_, z1_, wz_: (bi_[b], 0, z0_[b, z], 0, 0)),
            pl.BlockSpec((1, c, 1, h, w),
                         lambda b, z, bi_, z0_, z1_, wz_: (bi_[b], 0, z1_[b, z], 0, 0)),
        ],
        out_specs=pl.BlockSpec((1, 1, _CROP, c, _CROP),
                               lambda b, z, *sp: (b, z, 0, 0, 0)),
    )

    out = pl.pallas_call(
        _interp_kernel,
        grid_spec=grid_spec,
        out_shape=jax.ShapeDtypeStruct((nb, _CROP, _CROP, c, _CROP), jnp.float32),
        compiler_params=pltpu.CompilerParams(
            dimension_semantics=("arbitrary", "arbitrary"),
        ),
    )(bi, z0, z1, wz, wy, wxt, image, image)

    return jnp.transpose(out, (0, 3, 1, 2, 4))


# speedup vs baseline: 2.3569x; 1.0042x over previous
import jax
import jax.numpy as jnp
from jax.experimental import pallas as pl
from jax.experimental.pallas import tpu as pltpu

_CROP = 24


def _axis(lo, hi, size, crop):
    scale = (hi - lo) * (size - 1.0) / (crop - 1.0)
    c = lo[:, None] * (size - 1.0) + (
        jnp.arange(crop, dtype=jnp.float32)[None, :] * scale[:, None]
    )
    valid = (c >= 0.0) & (c <= size - 1.0)
    c0 = jnp.floor(c)
    frac = c - c0
    i0 = jnp.clip(c0.astype(jnp.int32), 0, size - 1)
    i1 = jnp.clip(i0 + 1, 0, size - 1)
    return i0, i1, frac, valid


def _weight_matrix(i0, i1, frac, valid, size):
    oh0 = jax.nn.one_hot(i0, size, dtype=jnp.float32)
    oh1 = jax.nn.one_hot(i1, size, dtype=jnp.float32)
    w = oh0 * (1.0 - frac)[..., None] + oh1 * frac[..., None]
    return w * valid.astype(jnp.float32)[..., None]


def _interp_kernel(bi_ref, z0_ref, z1_ref, wz_ref, wy_ref, wxt_ref,
                   p0_ref, p1_ref, out_ref):
    b = pl.program_id(0)
    z = pl.program_id(1)
    p0 = p0_ref[0, :, 0, :, :]
    p1 = p1_ref[0, :, 0, :, :]
    p = (p0 * wz_ref[b, z, 0] + p1 * wz_ref[b, z, 1]).astype(jnp.bfloat16)
    s = jax.lax.dot_general(p.reshape(32 * 64, 64), wxt_ref[0],
                            (((1,), (0,)), ((), ())),
                            preferred_element_type=jnp.float32)
    s4 = (s.reshape(32, 64, 24).astype(jnp.bfloat16)
          .transpose(1, 0, 2).reshape(64, 32 * 24))
    q = jax.lax.dot_general(wy_ref[0], s4, (((1,), (0,)), ((), ())),
                            preferred_element_type=jnp.float32)
    out_ref[0, 0] = q.reshape(24, 32, 24)


def kernel(image, boxes, box_ind):
    n, c, d, h, w = image.shape
    bz1, by1, bx1, bz2, by2, bx2 = (boxes[:, i] for i in range(6))
    z0, z1, fz, vz = _axis(bz1, bz2, d, _CROP)
    y0, y1, fy, vy = _axis(by1, by2, h, _CROP)
    x0, x1, fx, vx = _axis(bx1, bx2, w, _CROP)

    vzf = vz.astype(jnp.float32)
    wz = jnp.stack([(1.0 - fz) * vzf, fz * vzf], axis=-1)
    wy = _weight_matrix(y0, y1, fy, vy, h).astype(jnp.bfloat16)
    wxt = jnp.transpose(_weight_matrix(x0, x1, fx, vx, w),
                        (0, 2, 1)).astype(jnp.bfloat16)

    bi = box_ind.astype(jnp.int32)
    nb = boxes.shape[0]

    grid_spec = pltpu.PrefetchScalarGridSpec(
        num_scalar_prefetch=4,
        grid=(nb, _CROP),
        in_specs=[
            pl.BlockSpec((1, _CROP, h), lambda b, z, *sp: (b, 0, 0)),
            pl.BlockSpec((1, w, _CROP), lambda b, z, *sp: (b, 0, 0)),
            pl.BlockSpec((1, c, 1, h, w),
                         lambda b, z, bi_, z0_, z1_, wz_: (bi_[b], 0, z0_[b, z], 0, 0)),
            pl.BlockSpec((1, c, 1, h, w),
                         lambda b, z, bi_, z0_, z1_, wz_: (bi_[b], 0, z1_[b, z], 0, 0)),
        ],
        out_specs=pl.BlockSpec((1, 1, _CROP, c, _CROP),
                               lambda b, z, *sp: (b, z, 0, 0, 0)),
    )

    out = pl.pallas_call(
        _interp_kernel,
        grid_spec=grid_spec,
        out_shape=jax.ShapeDtypeStruct((nb, _CROP, _CROP, c, _CROP), jnp.float32),
        compiler_params=pltpu.CompilerParams(
            dimension_semantics=("arbitrary", "arbitrary"),
        ),
    )(bi, z0, z1, wz, wy, wxt, image, image)

    return jnp.transpose(out, (0, 3, 1, 2, 4))
